# baseline (speedup 1.0000x reference)
import functools

import jax
import jax.numpy as jnp
from jax import lax
from jax.experimental import pallas as pl
from jax.experimental.pallas import tpu as pltpu
from jax.experimental.pallas import tpu_sc as plsc

START_IND = 1024
END_IND = 1025

B = 8
LT = 130
TP = 256
TC = 514
NC = 8
D = 1024
V = 1026
S = 900

_NW = 32
_NB = 3


def _sc_body(text_ids, prom_ids, code_ids, ttab, wflat, pos, mvec,
             text_e, prom_g, code_g, mask_f,
             g0, g1, g2, o0, o1, o2, p0, p1, p2, i0, i1, i2, idsbuf, mall,
             mflat, sg0, sg1, sg2, sp0, sp1, sp2, so0, so1, so2, st):
    cid = lax.axis_index("c")
    sid = lax.axis_index("s")
    wid = sid * 2 + cid

    lanes = lax.iota(jnp.int32, 16)
    offs8 = lanes & 7

    gb = (g0, g1, g2)
    ob = (o0, o1, o2)
    pb = (p0, p1, p2)
    ib = (i0, i1, i2)
    sgb = (sg0, sg1, sg2)
    spb = (sp0, sp1, sp2)
    sob = (so0, so1, so2)

    def text_chunk(b, t0, n):
        pltpu.sync_copy(text_ids.at[b, pl.ds(t0, 16)], i0)
        pltpu.async_copy(ttab.at[i0], g0, st).wait()
        pltpu.sync_copy(pos.at[pl.ds(t0, 16), :], g1)

        def add_body(j, _):
            r = j >> 6
            cc = (j & 63) * 16
            g0[r, pl.ds(cc, 16)] = g0[r, pl.ds(cc, 16)] + g1[r, pl.ds(cc, 16)]
            return 0

        lax.fori_loop(0, n * 64, add_body, 0)
        if n == 16:
            pltpu.sync_copy(g0, text_e.at[b, pl.ds(t0, 16), :])
        else:
            pltpu.sync_copy(g0.at[pl.ds(0, 2), :], text_e.at[b, pl.ds(t0, 2), :])

    def text_loop(k, _):
        c = wid + k * 32
        text_chunk(c >> 3, (c & 7) * 16, 16)
        return 0

    lax.fori_loop(0, 2, text_loop, 0)

    @pl.when(wid < 8)
    def _():
        text_chunk(wid, 128, 2)

    def segment(ids_flat, out, T, start, n, nids):
        npb = T // 2
        n = jnp.int32(n)

        def pair_coords(k):
            p = start + k
            b = p // npb
            t = 2 * (p - b * npb)
            return b, t

        pltpu.sync_copy(ids_flat.at[pl.ds(16 * start, nids)],
                        idsbuf.at[pl.ds(0, nids)])

        def issue(k, par):
            b, t = pair_coords(k)
            ib[par][...] = (idsbuf[pl.ds(16 * k, 16)] << 3) + offs8
            pltpu.async_copy(wflat.at[ib[par]], gb[par], sgb[par])
            pltpu.async_copy(pos.at[pl.ds(t, 2), :], pb[par], spb[par])

        def drain_out(k, par):
            b, t = pair_coords(k)
            pltpu.make_async_copy(ob[par], out.at[b, pl.ds(t, 2)],
                                  sob[par]).wait()

        def process(k, par):
            b, t = pair_coords(k)
            pltpu.make_async_copy(wflat.at[ib[par]], gb[par], sgb[par]).wait()
            pltpu.make_async_copy(pos.at[pl.ds(t, 2), :], pb[par],
                                  spb[par]).wait()

            @pl.when(k >= _NB)
            def _():
                drain_out(k - _NB, par)

            rows = gb[par]
            pvs = pb[par]
            obuf = ob[par]

            def addp(j, _):
                dh = j >> 3
                cc = (j & 7) * 16
                for tt in range(2):
                    pv = pvs[tt, pl.ds(dh * 128 + cc, 16)]
                    for i in range(8):
                        obuf[tt, dh, i, pl.ds(cc, 16)] = (
                            rows[8 * tt + i, pl.ds(dh * 128 + cc, 16)] + pv)
                return 0

            lax.fori_loop(0, 64, addp, 0)
            pltpu.async_copy(obuf, out.at[b, pl.ds(t, 2)], sob[par])

        @pl.when(n > 0)
        def _():
            issue(0, 0)

        def outer(kk, _):
            for par in range(_NB):
                k = kk * _NB + par

                @pl.when(k < n)
                def _(k=k, par=par):
                    kn = k + 1
                    parn = (par + 1) % _NB

                    @pl.when(kn < n)
                    def _():
                        issue(kn, parn)

                    process(k, par)
            return 0

        lax.fori_loop(0, (n + _NB - 1) // _NB, outer, 0)

        for par in range(_NB):
            klast = jnp.where((n - 1) % _NB == par, n - 1,
                              jnp.where((n - 2) % _NB == par, n - 2, n - 3))

            @pl.when(klast >= 0)
            def _(klast=klast, par=par):
                drain_out(klast, par)

    segment(prom_ids, prom_g, TP, wid * 32, 32, 512)
    ncode = B * (TC // 2)
    nc_extra = ncode - 32 * (ncode // 32)
    code_start = wid * (ncode // 32) + jnp.minimum(wid, nc_extra)
    code_n = ncode // 32 + jnp.where(wid < nc_extra, 1, 0)
    segment(code_ids, code_g, TC, code_start, code_n, 1056)

    pltpu.sync_copy(mvec, mall)
    nmask = (B * S * S) // 3600
    nmask_mine = jnp.where(wid < nmask - 32 * (nmask // 32), nmask // 32 + 1,
                           nmask // 32)
    lane4 = lanes & 3

    def mask_chunk(k, _):
        ch = wid + k * 32
        a4 = ch * 4
        base = ch * 3600
        gr = a4 + lane4
        bv = gr // S
        rv = gr - bv * S
        mv16 = plsc.load_gather(mall, [bv, rv])
        s0 = jnp.full((16,), mv16[0], jnp.float32)
        s1 = jnp.full((16,), mv16[1], jnp.float32)
        s2 = jnp.full((16,), mv16[2], jnp.float32)
        s3 = jnp.full((16,), mv16[3], jnp.float32)
        b0 = a4 // S

        def mvreg(u, _):
            fl = u * 16 + lanes
            rowv = (jnp.where(fl >= S, 1, 0) + jnp.where(fl >= 2 * S, 1, 0)
                    + jnp.where(fl >= 3 * S, 1, 0))
            colv = fl - rowv * S
            blv = jnp.where(a4 + rowv >= (b0 + 1) * S, b0 + 1, b0)
            av = jnp.where(fl < S, s0,
                           jnp.where(fl < 2 * S, s1,
                                     jnp.where(fl < 3 * S, s2, s3)))
            bb = plsc.load_gather(mall, [blv, colv])
            mflat[pl.ds(u * 16, 16)] = av * bb
            return 0

        lax.fori_loop(0, 225, mvreg, 0)
        pltpu.sync_copy(mflat, mask_f.at[pl.ds(base, 3600)])
        return 0

    lax.fori_loop(0, nmask_mine, mask_chunk, 0)


@jax.jit
def _sc_call(text_ids_pad, prom_ids_f, code_ids_f, text_table, wflat, pos,
             mvec):
    mesh = plsc.VectorSubcoreMesh(core_axis_name="c", subcore_axis_name="s",
                                  num_cores=2, num_subcores=16)
    f = pl.kernel(
        _sc_body,
        mesh=mesh,
        compiler_params=pltpu.CompilerParams(use_tc_tiling_on_sc=False,
                                             needs_layout_passes=False),
        out_type=[
            jax.ShapeDtypeStruct((B, LT, D), jnp.float32),
            jax.ShapeDtypeStruct((B, TP, 8, NC, 128), jnp.float32),
            jax.ShapeDtypeStruct((B, TC, 8, NC, 128), jnp.float32),
            jax.ShapeDtypeStruct((B * S * S,), jnp.float32),
        ],
        scratch_types=[
            pltpu.VMEM((16, D), jnp.float32),
            pltpu.VMEM((16, D), jnp.float32),
            pltpu.VMEM((16, D), jnp.float32),
            pltpu.VMEM((2, 8, NC, 128), jnp.float32),
            pltpu.VMEM((2, 8, NC, 128), jnp.float32),
            pltpu.VMEM((2, 8, NC, 128), jnp.float32),
            pltpu.VMEM((2, D), jnp.float32),
            pltpu.VMEM((2, D), jnp.float32),
            pltpu.VMEM((2, D), jnp.float32),
            pltpu.VMEM((16,), jnp.int32),
            pltpu.VMEM((16,), jnp.int32),
            pltpu.VMEM((16,), jnp.int32),
            pltpu.VMEM((1056,), jnp.int32),
            pltpu.VMEM((B, 1024), jnp.float32),
            pltpu.VMEM((3600,), jnp.float32),
            pltpu.SemaphoreType.DMA,
            pltpu.SemaphoreType.DMA,
            pltpu.SemaphoreType.DMA,
            pltpu.SemaphoreType.DMA,
            pltpu.SemaphoreType.DMA,
            pltpu.SemaphoreType.DMA,
            pltpu.SemaphoreType.DMA,
            pltpu.SemaphoreType.DMA,
            pltpu.SemaphoreType.DMA,
            pltpu.SemaphoreType.DMA,
        ],
    )
    return f(text_ids_pad, prom_ids_f, code_ids_f, text_table, wflat, pos,
             mvec)


def kernel(text, prom, code, text_table, wave_tables, pos_emb):
    text = jnp.where(text == -1, END_IND, text)
    prom_ids = jnp.where(prom == -1, END_IND, prom)
    code_in = jnp.where(code == -1, END_IND, code)
    text_ids = jnp.pad(text, ((0, 0), (1, 0)), constant_values=START_IND)
    text_ids = jnp.pad(text_ids, ((0, 0), (0, 1)), constant_values=END_IND)
    code_ids = jnp.pad(code_in, ((0, 0), (1, 0), (0, 0)),
                       constant_values=START_IND)
    code_ids = jnp.pad(code_ids, ((0, 0), (0, 1), (0, 0)),
                       constant_values=END_IND)
    gt = code_ids

    text_mask = jnp.pad((text_ids != END_IND)[:, :-1], ((0, 0), (1, 0)),
                        constant_values=True)
    prom_mask = jnp.pad((prom_ids != END_IND)[:, :-1, 0], ((0, 0), (1, 0)),
                        constant_values=True)
    code_mask = jnp.pad((code_ids != END_IND)[:, :-1, 0], ((0, 0), (1, 0)),
                        constant_values=True)
    m = jnp.concatenate((text_mask, prom_mask, code_mask), axis=1
                        ).astype(jnp.float32)
    gt_mask = code_mask

    text_ids_pad = jnp.pad(text_ids, ((0, 0), (0, 144 - LT)))
    prom_ids_f = prom_ids.reshape(B * TP * NC)
    code_ids_f = jnp.pad(code_ids.reshape(B * TC * NC), (0, 33024 - B * TC * NC))
    wflat = jnp.transpose(wave_tables, (1, 0, 2)).reshape(V * NC, D)
    mvec = jnp.pad(m, ((0, 0), (0, 1024 - S)))

    text_e, prom_g, code_g, mask_f = _sc_call(
        text_ids_pad, prom_ids_f, code_ids_f, text_table, wflat,
        pos_emb[:TC + 6], mvec)

    prom_e = prom_g.transpose(0, 1, 2, 4, 3).reshape(B, TP, D, NC)
    code_e = code_g.transpose(0, 1, 2, 4, 3).reshape(B, TC, D, NC)
    mask = mask_f.reshape(B, S, S)
    return (text_e, prom_e, code_e, mask, gt, gt_mask)


# speedup vs baseline: 1.6501x; 1.6501x over previous
import functools

import jax
import jax.numpy as jnp
from jax import lax
from jax.experimental import pallas as pl
from jax.experimental.pallas import tpu as pltpu
from jax.experimental.pallas import tpu_sc as plsc

START_IND = 1024
END_IND = 1025

B = 8
LT = 130
TP = 256
TC = 514
NC = 8
D = 1024
V = 1026
S = 900

_NW = 32
_NB = 3


def _sc_body(text_ids, prom_ids, code_ids, ttab, wflat, pos, mvec,
             text_e, prom_g, code_g, mask_f,
             g0, g1, g2, o0, o1, o2, p0, p1, p2, i0, i1, i2, idsbuf, mall,
             mflat, sg0, sg1, sg2, sp0, sp1, sp2, so0, so1, so2, st):
    cid = lax.axis_index("c")
    sid = lax.axis_index("s")
    wid = sid * 2 + cid

    lanes = lax.iota(jnp.int32, 16)
    offs8 = lanes & 7

    gb = (g0, g1, g2)
    ob = (o0, o1, o2)
    pb = (p0, p1, p2)
    ib = (i0, i1, i2)
    sgb = (sg0, sg1, sg2)
    spb = (sp0, sp1, sp2)
    sob = (so0, so1, so2)

    def text_chunk(b, t0, n):
        pltpu.sync_copy(text_ids.at[b, pl.ds(t0, 16)], i0)
        pltpu.async_copy(ttab.at[i0], g0, st).wait()
        pltpu.sync_copy(pos.at[pl.ds(t0, 16), :], g1)

        def add_body(j, _):
            r = j >> 6
            cc = (j & 63) * 16
            g0[r, pl.ds(cc, 16)] = g0[r, pl.ds(cc, 16)] + g1[r, pl.ds(cc, 16)]
            return 0

        lax.fori_loop(0, n * 64, add_body, 0)
        if n == 16:
            pltpu.sync_copy(g0, text_e.at[b, pl.ds(t0, 16), :])
        else:
            pltpu.sync_copy(g0.at[pl.ds(0, 2), :], text_e.at[b, pl.ds(t0, 2), :])

    def text_loop(k, _):
        c = wid + k * 32
        text_chunk(c >> 3, (c & 7) * 16, 16)
        return 0

    lax.fori_loop(0, 2, text_loop, 0)

    @pl.when(wid < 8)
    def _():
        text_chunk(wid, 128, 2)

    def segment(ids_flat, out, T, start, n, nids):
        npb = T // 2
        n = jnp.int32(n)

        def pair_coords(k):
            p = start + k
            b = p // npb
            t = 2 * (p - b * npb)
            return b, t

        pltpu.sync_copy(ids_flat.at[pl.ds(16 * start, nids)],
                        idsbuf.at[pl.ds(0, nids)])

        def issue(k, par):
            b, t = pair_coords(k)
            ib[par][...] = (idsbuf[pl.ds(16 * k, 16)] << 3) + offs8
            pltpu.async_copy(wflat.at[ib[par]], gb[par], sgb[par])
            pltpu.async_copy(pos.at[pl.ds(t, 2), :], pb[par], spb[par])

        def drain_out(k, par):
            b, t = pair_coords(k)
            pltpu.make_async_copy(ob[par], out.at[b, pl.ds(t, 2)],
                                  sob[par]).wait()

        def process(k, par):
            b, t = pair_coords(k)
            pltpu.make_async_copy(wflat.at[ib[par]], gb[par], sgb[par]).wait()
            pltpu.make_async_copy(pos.at[pl.ds(t, 2), :], pb[par],
                                  spb[par]).wait()

            @pl.when(k >= _NB)
            def _():
                drain_out(k - _NB, par)

            rows = gb[par]
            pvs = pb[par]
            obuf = ob[par]

            def addp(c, _):
                cc = c * 16
                for tt in range(2):
                    for dh in range(8):
                        pv = pvs[tt, pl.ds(dh * 128 + cc, 16)]
                        for i in range(8):
                            obuf[tt, dh, i, pl.ds(cc, 16)] = (
                                rows[8 * tt + i, pl.ds(dh * 128 + cc, 16)]
                                + pv)
                return 0

            lax.fori_loop(0, 8, addp, 0)
            pltpu.async_copy(obuf, out.at[b, pl.ds(t, 2)], sob[par])

        @pl.when(n > 0)
        def _():
            issue(0, 0)

        def outer(kk, _):
            for par in range(_NB):
                k = kk * _NB + par

                @pl.when(k < n)
                def _(k=k, par=par):
                    kn = k + 1
                    parn = (par + 1) % _NB

                    @pl.when(kn < n)
                    def _():
                        issue(kn, parn)

                    process(k, par)
            return 0

        lax.fori_loop(0, (n + _NB - 1) // _NB, outer, 0)

        for par in range(_NB):
            klast = jnp.where((n - 1) % _NB == par, n - 1,
                              jnp.where((n - 2) % _NB == par, n - 2, n - 3))

            @pl.when(klast >= 0)
            def _(klast=klast, par=par):
                drain_out(klast, par)

    segment(prom_ids, prom_g, TP, wid * 32, 32, 512)
    ncode = B * (TC // 2)
    nc_extra = ncode - 32 * (ncode // 32)
    code_start = wid * (ncode // 32) + jnp.minimum(wid, nc_extra)
    code_n = ncode // 32 + jnp.where(wid < nc_extra, 1, 0)
    segment(code_ids, code_g, TC, code_start, code_n, 1056)

    pltpu.sync_copy(mvec, mall)
    nmask = (B * S * S) // 3600
    nmask_mine = jnp.where(wid < nmask - 32 * (nmask // 32), nmask // 32 + 1,
                           nmask // 32)
    lane4 = lanes & 3

    def mask_chunk(k, _):
        ch = wid + k * 32
        a4 = ch * 4
        base = ch * 3600
        gr = a4 + lane4
        bv = gr // S
        rv = gr - bv * S
        mv16 = plsc.load_gather(mall, [bv, rv])
        s0 = jnp.full((16,), mv16[0], jnp.float32)
        s1 = jnp.full((16,), mv16[1], jnp.float32)
        s2 = jnp.full((16,), mv16[2], jnp.float32)
        s3 = jnp.full((16,), mv16[3], jnp.float32)
        b0 = a4 // S

        def mvreg(u, _):
            fl = u * 16 + lanes
            rowv = (jnp.where(fl >= S, 1, 0) + jnp.where(fl >= 2 * S, 1, 0)
                    + jnp.where(fl >= 3 * S, 1, 0))
            colv = fl - rowv * S
            blv = jnp.where(a4 + rowv >= (b0 + 1) * S, b0 + 1, b0)
            av = jnp.where(fl < S, s0,
                           jnp.where(fl < 2 * S, s1,
                                     jnp.where(fl < 3 * S, s2, s3)))
            bb = plsc.load_gather(mall, [blv, colv])
            mflat[pl.ds(u * 16, 16)] = av * bb
            return 0

        lax.fori_loop(0, 225, mvreg, 0)
        pltpu.sync_copy(mflat, mask_f.at[pl.ds(base, 3600)])
        return 0

    lax.fori_loop(0, nmask_mine, mask_chunk, 0)


@jax.jit
def _sc_call(text_ids_pad, prom_ids_f, code_ids_f, text_table, wflat, pos,
             mvec):
    mesh = plsc.VectorSubcoreMesh(core_axis_name="c", subcore_axis_name="s",
                                  num_cores=2, num_subcores=16)
    f = pl.kernel(
        _sc_body,
        mesh=mesh,
        compiler_params=pltpu.CompilerParams(use_tc_tiling_on_sc=False,
                                             needs_layout_passes=False),
        out_type=[
            jax.ShapeDtypeStruct((B, LT, D), jnp.float32),
            jax.ShapeDtypeStruct((B, TP, 8, NC, 128), jnp.float32),
            jax.ShapeDtypeStruct((B, TC, 8, NC, 128), jnp.float32),
            jax.ShapeDtypeStruct((B * S * S,), jnp.float32),
        ],
        scratch_types=[
            pltpu.VMEM((16, D), jnp.float32),
            pltpu.VMEM((16, D), jnp.float32),
            pltpu.VMEM((16, D), jnp.float32),
            pltpu.VMEM((2, 8, NC, 128), jnp.float32),
            pltpu.VMEM((2, 8, NC, 128), jnp.float32),
            pltpu.VMEM((2, 8, NC, 128), jnp.float32),
            pltpu.VMEM((2, D), jnp.float32),
            pltpu.VMEM((2, D), jnp.float32),
            pltpu.VMEM((2, D), jnp.float32),
            pltpu.VMEM((16,), jnp.int32),
            pltpu.VMEM((16,), jnp.int32),
            pltpu.VMEM((16,), jnp.int32),
            pltpu.VMEM((1056,), jnp.int32),
            pltpu.VMEM((B, 1024), jnp.float32),
            pltpu.VMEM((3600,), jnp.float32),
            pltpu.SemaphoreType.DMA,
            pltpu.SemaphoreType.DMA,
            pltpu.SemaphoreType.DMA,
            pltpu.SemaphoreType.DMA,
            pltpu.SemaphoreType.DMA,
            pltpu.SemaphoreType.DMA,
            pltpu.SemaphoreType.DMA,
            pltpu.SemaphoreType.DMA,
            pltpu.SemaphoreType.DMA,
            pltpu.SemaphoreType.DMA,
        ],
    )
    return f(text_ids_pad, prom_ids_f, code_ids_f, text_table, wflat, pos,
             mvec)


def kernel(text, prom, code, text_table, wave_tables, pos_emb):
    text = jnp.where(text == -1, END_IND, text)
    prom_ids = jnp.where(prom == -1, END_IND, prom)
    code_in = jnp.where(code == -1, END_IND, code)
    text_ids = jnp.pad(text, ((0, 0), (1, 0)), constant_values=START_IND)
    text_ids = jnp.pad(text_ids, ((0, 0), (0, 1)), constant_values=END_IND)
    code_ids = jnp.pad(code_in, ((0, 0), (1, 0), (0, 0)),
                       constant_values=START_IND)
    code_ids = jnp.pad(code_ids, ((0, 0), (0, 1), (0, 0)),
                       constant_values=END_IND)
    gt = code_ids

    text_mask = jnp.pad((text_ids != END_IND)[:, :-1], ((0, 0), (1, 0)),
                        constant_values=True)
    prom_mask = jnp.pad((prom_ids != END_IND)[:, :-1, 0], ((0, 0), (1, 0)),
                        constant_values=True)
    code_mask = jnp.pad((code_ids != END_IND)[:, :-1, 0], ((0, 0), (1, 0)),
                        constant_values=True)
    m = jnp.concatenate((text_mask, prom_mask, code_mask), axis=1
                        ).astype(jnp.float32)
    gt_mask = code_mask

    text_ids_pad = jnp.pad(text_ids, ((0, 0), (0, 144 - LT)))
    prom_ids_f = prom_ids.reshape(B * TP * NC)
    code_ids_f = jnp.pad(code_ids.reshape(B * TC * NC), (0, 33024 - B * TC * NC))
    wflat = jnp.transpose(wave_tables, (1, 0, 2)).reshape(V * NC, D)
    mvec = jnp.pad(m, ((0, 0), (0, 1024 - S)))

    text_e, prom_g, code_g, mask_f = _sc_call(
        text_ids_pad, prom_ids_f, code_ids_f, text_table, wflat,
        pos_emb[:TC + 6], mvec)

    prom_e = prom_g.transpose(0, 1, 2, 4, 3).reshape(B, TP, D, NC)
    code_e = code_g.transpose(0, 1, 2, 4, 3).reshape(B, TC, D, NC)
    mask = mask_f.reshape(B, S, S)
    return (text_e, prom_e, code_e, mask, gt, gt_mask)


# speedup vs baseline: 1.9785x; 1.1990x over previous
import functools

import jax
import jax.numpy as jnp
from jax import lax
from jax.experimental import pallas as pl
from jax.experimental.pallas import tpu as pltpu
from jax.experimental.pallas import tpu_sc as plsc

START_IND = 1024
END_IND = 1025

B = 8
LT = 130
TP = 256
TC = 514
NC = 8
D = 1024
V = 1026
S = 900

_NW = 32
_NB = 3


def _sc_mt_body(text_ids, ttab, pos, mvec,
                text_e, mask_f,
                g0, g1, i0, mall, mflat, st):
    cid = lax.axis_index("c")
    sid = lax.axis_index("s")
    wid = sid * 2 + cid

    lanes = lax.iota(jnp.int32, 16)

    def text_chunk(b, t0, n):
        pltpu.sync_copy(text_ids.at[b, pl.ds(t0, 16)], i0)
        pltpu.async_copy(ttab.at[i0], g0, st).wait()
        pltpu.sync_copy(pos.at[pl.ds(t0, 16), :], g1)

        def add_body(j, _):
            r = j >> 6
            cc = (j & 63) * 16
            g0[r, pl.ds(cc, 16)] = g0[r, pl.ds(cc, 16)] + g1[r, pl.ds(cc, 16)]
            return 0

        lax.fori_loop(0, n * 64, add_body, 0)
        if n == 16:
            pltpu.sync_copy(g0, text_e.at[b, pl.ds(t0, 16), :])
        else:
            pltpu.sync_copy(g0.at[pl.ds(0, 2), :], text_e.at[b, pl.ds(t0, 2), :])

    def text_loop(k, _):
        c = wid + k * 32
        text_chunk(c >> 3, (c & 7) * 16, 16)
        return 0

    lax.fori_loop(0, 2, text_loop, 0)

    @pl.when(wid < 8)
    def _():
        text_chunk(wid, 128, 2)

    pltpu.sync_copy(mvec, mall)
    nmask = (B * S * S) // 3600
    nmask_mine = jnp.where(wid < nmask - 32 * (nmask // 32), nmask // 32 + 1,
                           nmask // 32)
    lane4 = lanes & 3

    def mask_chunk(k, _):
        ch = wid + k * 32
        a4 = ch * 4
        base = ch * 3600
        gr = a4 + lane4
        bv = gr // S
        rv = gr - bv * S
        mv16 = plsc.load_gather(mall, [bv, rv])
        s0 = jnp.full((16,), mv16[0], jnp.float32)
        s1 = jnp.full((16,), mv16[1], jnp.float32)
        s2 = jnp.full((16,), mv16[2], jnp.float32)
        s3 = jnp.full((16,), mv16[3], jnp.float32)
        b0 = a4 // S

        def mvreg(u, _):
            fl = u * 16 + lanes
            rowv = (jnp.where(fl >= S, 1, 0) + jnp.where(fl >= 2 * S, 1, 0)
                    + jnp.where(fl >= 3 * S, 1, 0))
            colv = fl - rowv * S
            blv = jnp.where(a4 + rowv >= (b0 + 1) * S, b0 + 1, b0)
            av = jnp.where(fl < S, s0,
                           jnp.where(fl < 2 * S, s1,
                                     jnp.where(fl < 3 * S, s2, s3)))
            bb = plsc.load_gather(mall, [blv, colv])
            mflat[pl.ds(u * 16, 16)] = av * bb
            return 0

        lax.fori_loop(0, 225, mvreg, 0)
        pltpu.sync_copy(mflat, mask_f.at[pl.ds(base, 3600)])
        return 0

    lax.fori_loop(0, nmask_mine, mask_chunk, 0)


def _sc_wave_body(prom_ids, code_ids, wflat, pos,
                  prom_g, code_g,
                  g0, g1, g2, o0, o1, o2, p0, p1, p2, i0, i1, i2, idsbuf,
                  sg0, sg1, sg2, sp0, sp1, sp2, so0, so1, so2):
    cid = lax.axis_index("c")
    sid = lax.axis_index("s")
    wid = sid * 2 + cid

    lanes = lax.iota(jnp.int32, 16)
    offs8 = lanes & 7

    gb = (g0, g1, g2)
    ob = (o0, o1, o2)
    pb = (p0, p1, p2)
    ib = (i0, i1, i2)
    sgb = (sg0, sg1, sg2)
    spb = (sp0, sp1, sp2)
    sob = (so0, so1, so2)

    def segment(ids_flat, out, T, start, n, nids):
        npb = T // 2
        n = jnp.int32(n)

        def pair_coords(k):
            p = start + k
            b = p // npb
            t = 2 * (p - b * npb)
            return b, t

        pltpu.sync_copy(ids_flat.at[pl.ds(16 * start, nids)],
                        idsbuf.at[pl.ds(0, nids)])

        def issue(k, par):
            b, t = pair_coords(k)
            ib[par][...] = (idsbuf[pl.ds(16 * k, 16)] << 3) + offs8
            pltpu.async_copy(wflat.at[ib[par]], gb[par], sgb[par])
            pltpu.async_copy(pos.at[pl.ds(t, 2), :], pb[par], spb[par])

        def drain_out(k, par):
            b, t = pair_coords(k)
            pltpu.make_async_copy(ob[par], out.at[b, pl.ds(t, 2)],
                                  sob[par]).wait()

        def process(k, par):
            b, t = pair_coords(k)
            pltpu.make_async_copy(wflat.at[ib[par]], gb[par], sgb[par]).wait()
            pltpu.make_async_copy(pos.at[pl.ds(t, 2), :], pb[par],
                                  spb[par]).wait()

            @pl.when(k >= _NB)
            def _():
                drain_out(k - _NB, par)

            rows = gb[par]
            pvs = pb[par]
            obuf = ob[par]

            def addp(c, _):
                cc = c * 16
                for tt in range(2):
                    for dh in range(8):
                        pv = pvs[tt, pl.ds(dh * 128 + cc, 16)]
                        for i in range(8):
                            obuf[tt, dh, i, pl.ds(cc, 16)] = (
                                rows[8 * tt + i, pl.ds(dh * 128 + cc, 16)]
                                + pv)
                return 0

            lax.fori_loop(0, 8, addp, 0)
            pltpu.async_copy(obuf, out.at[b, pl.ds(t, 2)], sob[par])

        @pl.when(n > 0)
        def _():
            issue(0, 0)

        def outer(kk, _):
            for par in range(_NB):
                k = kk * _NB + par

                @pl.when(k < n)
                def _(k=k, par=par):
                    kn = k + 1
                    parn = (par + 1) % _NB

                    @pl.when(kn < n)
                    def _():
                        issue(kn, parn)

                    process(k, par)
            return 0

        lax.fori_loop(0, (n + _NB - 1) // _NB, outer, 0)

        for par in range(_NB):
            klast = jnp.where((n - 1) % _NB == par, n - 1,
                              jnp.where((n - 2) % _NB == par, n - 2, n - 3))

            @pl.when(klast >= 0)
            def _(klast=klast, par=par):
                drain_out(klast, par)

    segment(prom_ids, prom_g, TP, wid * 32, 32, 512)
    ncode = B * (TC // 2)
    nc_extra = ncode - 32 * (ncode // 32)
    code_start = wid * (ncode // 32) + jnp.minimum(wid, nc_extra)
    code_n = ncode // 32 + jnp.where(wid < nc_extra, 1, 0)
    segment(code_ids, code_g, TC, code_start, code_n, 1056)


@jax.jit
def _sc_call(text_ids_pad, prom_ids_f, code_ids_f, text_table, wflat, pos,
             mvec):
    mesh = plsc.VectorSubcoreMesh(core_axis_name="c", subcore_axis_name="s",
                                  num_cores=2, num_subcores=16)
    params = pltpu.CompilerParams(use_tc_tiling_on_sc=False,
                                  needs_layout_passes=False)
    mt = pl.kernel(
        _sc_mt_body,
        mesh=mesh,
        compiler_params=params,
        out_type=[
            jax.ShapeDtypeStruct((B, LT, D), jnp.float32),
            jax.ShapeDtypeStruct((B * S * S,), jnp.float32),
        ],
        scratch_types=[
            pltpu.VMEM((16, D), jnp.float32),
            pltpu.VMEM((16, D), jnp.float32),
            pltpu.VMEM((16,), jnp.int32),
            pltpu.VMEM((B, 1024), jnp.float32),
            pltpu.VMEM((3600,), jnp.float32),
            pltpu.SemaphoreType.DMA,
        ],
    )
    text_e, mask_f = mt(text_ids_pad, text_table, pos, mvec)

    wave = pl.kernel(
        _sc_wave_body,
        mesh=mesh,
        compiler_params=params,
        out_type=[
            jax.ShapeDtypeStruct((B, TP, 8, NC, 128), jnp.float32),
            jax.ShapeDtypeStruct((B, TC, 8, NC, 128), jnp.float32),
        ],
        scratch_types=[
            pltpu.VMEM((16, D), jnp.float32),
            pltpu.VMEM((16, D), jnp.float32),
            pltpu.VMEM((16, D), jnp.float32),
            pltpu.VMEM((2, 8, NC, 128), jnp.float32),
            pltpu.VMEM((2, 8, NC, 128), jnp.float32),
            pltpu.VMEM((2, 8, NC, 128), jnp.float32),
            pltpu.VMEM((2, D), jnp.float32),
            pltpu.VMEM((2, D), jnp.float32),
            pltpu.VMEM((2, D), jnp.float32),
            pltpu.VMEM((16,), jnp.int32),
            pltpu.VMEM((16,), jnp.int32),
            pltpu.VMEM((16,), jnp.int32),
            pltpu.VMEM((1056,), jnp.int32),
            pltpu.SemaphoreType.DMA,
            pltpu.SemaphoreType.DMA,
            pltpu.SemaphoreType.DMA,
            pltpu.SemaphoreType.DMA,
            pltpu.SemaphoreType.DMA,
            pltpu.SemaphoreType.DMA,
            pltpu.SemaphoreType.DMA,
            pltpu.SemaphoreType.DMA,
            pltpu.SemaphoreType.DMA,
        ],
    )
    prom_g, code_g = wave(prom_ids_f, code_ids_f, wflat, pos)
    return text_e, prom_g, code_g, mask_f


def kernel(text, prom, code, text_table, wave_tables, pos_emb):
    text = jnp.where(text == -1, END_IND, text)
    prom_ids = jnp.where(prom == -1, END_IND, prom)
    code_in = jnp.where(code == -1, END_IND, code)
    text_ids = jnp.pad(text, ((0, 0), (1, 0)), constant_values=START_IND)
    text_ids = jnp.pad(text_ids, ((0, 0), (0, 1)), constant_values=END_IND)
    code_ids = jnp.pad(code_in, ((0, 0), (1, 0), (0, 0)),
                       constant_values=START_IND)
    code_ids = jnp.pad(code_ids, ((0, 0), (0, 1), (0, 0)),
                       constant_values=END_IND)
    gt = code_ids

    text_mask = jnp.pad((text_ids != END_IND)[:, :-1], ((0, 0), (1, 0)),
                        constant_values=True)
    prom_mask = jnp.pad((prom_ids != END_IND)[:, :-1, 0], ((0, 0), (1, 0)),
                        constant_values=True)
    code_mask = jnp.pad((code_ids != END_IND)[:, :-1, 0], ((0, 0), (1, 0)),
                        constant_values=True)
    m = jnp.concatenate((text_mask, prom_mask, code_mask), axis=1
                        ).astype(jnp.float32)
    gt_mask = code_mask

    text_ids_pad = jnp.pad(text_ids, ((0, 0), (0, 144 - LT)))
    prom_ids_f = prom_ids.reshape(B * TP * NC)
    code_ids_f = jnp.pad(code_ids.reshape(B * TC * NC), (0, 33024 - B * TC * NC))
    wflat = jnp.transpose(wave_tables, (1, 0, 2)).reshape(V * NC, D)
    mvec = jnp.pad(m, ((0, 0), (0, 1024 - S)))

    text_e, prom_g, code_g, mask_f = _sc_call(
        text_ids_pad, prom_ids_f, code_ids_f, text_table, wflat,
        pos_emb[:TC + 6], mvec)

    prom_e = prom_g.transpose(0, 1, 2, 4, 3).reshape(B, TP, D, NC)
    code_e = code_g.transpose(0, 1, 2, 4, 3).reshape(B, TC, D, NC)
    mask = mask_f.reshape(B, S, S)
    return (text_e, prom_e, code_e, mask, gt, gt_mask)


# speedup vs baseline: 2.2421x; 1.1333x over previous
import functools

import jax
import jax.numpy as jnp
from jax import lax
from jax.experimental import pallas as pl
from jax.experimental.pallas import tpu as pltpu
from jax.experimental.pallas import tpu_sc as plsc

START_IND = 1024
END_IND = 1025

B = 8
LT = 130
TP = 256
TC = 514
NC = 8
D = 1024
V = 1026
S = 900

_NW = 32
_NB = 3


def _sc_mt_body(text_ids, ttab, pos, mvec,
                text_e, mask_f,
                g0, g1, i0, mall, mflat, st):
    cid = lax.axis_index("c")
    sid = lax.axis_index("s")
    wid = sid * 2 + cid

    lanes = lax.iota(jnp.int32, 16)

    def text_chunk(b, t0, n):
        pltpu.sync_copy(text_ids.at[b, pl.ds(t0, 16)], i0)
        pltpu.async_copy(ttab.at[i0], g0, st).wait()
        pltpu.sync_copy(pos.at[pl.ds(t0, 16), :], g1)

        def add_body(j, _):
            r = j >> 6
            cc = (j & 63) * 16
            g0[r, pl.ds(cc, 16)] = g0[r, pl.ds(cc, 16)] + g1[r, pl.ds(cc, 16)]
            return 0

        lax.fori_loop(0, n * 64, add_body, 0)
        if n == 16:
            pltpu.sync_copy(g0, text_e.at[b, pl.ds(t0, 16), :])
        else:
            pltpu.sync_copy(g0.at[pl.ds(0, 2), :], text_e.at[b, pl.ds(t0, 2), :])

    def text_loop(k, _):
        c = wid + k * 32
        text_chunk(c >> 3, (c & 7) * 16, 16)
        return 0

    lax.fori_loop(0, 2, text_loop, 0)

    @pl.when(wid < 8)
    def _():
        text_chunk(wid, 128, 2)

    pltpu.sync_copy(mvec, mall)
    nmask = (B * S * S) // 3600
    nmask_mine = jnp.where(wid < nmask - 32 * (nmask // 32), nmask // 32 + 1,
                           nmask // 32)
    lane4 = lanes & 3

    def mask_chunk(k, _):
        ch = wid + k * 32
        a4 = ch * 4
        base = ch * 3600
        rbv = a4 + lane4
        mv16 = plsc.load_gather(mall, [rbv & 7, rbv >> 3])
        s0 = jnp.full((16,), mv16[0], jnp.float32)
        s1 = jnp.full((16,), mv16[1], jnp.float32)
        s2 = jnp.full((16,), mv16[2], jnp.float32)
        s3 = jnp.full((16,), mv16[3], jnp.float32)

        def mvreg(u, _):
            fl = u * 16 + lanes
            rowv = (jnp.where(fl >= S, 1, 0) + jnp.where(fl >= 2 * S, 1, 0)
                    + jnp.where(fl >= 3 * S, 1, 0))
            colv = fl - rowv * S
            blv = (a4 + rowv) & 7
            av = jnp.where(fl < S, s0,
                           jnp.where(fl < 2 * S, s1,
                                     jnp.where(fl < 3 * S, s2, s3)))
            bb = plsc.load_gather(mall, [blv, colv])
            mflat[pl.ds(u * 16, 16)] = av * bb
            return 0

        lax.fori_loop(0, 225, mvreg, 0)
        pltpu.sync_copy(mflat, mask_f.at[pl.ds(base, 3600)])
        return 0

    lax.fori_loop(0, nmask_mine, mask_chunk, 0)


def _sc_wave_body(prom_ids, code_ids, wflat, pos,
                  prom_g, code_g,
                  g0, g1, g2, o0, o1, o2, p0, p1, p2, i0, i1, i2, idsbuf,
                  sg0, sg1, sg2, sp0, sp1, sp2, so0, so1, so2):
    cid = lax.axis_index("c")
    sid = lax.axis_index("s")
    wid = sid * 2 + cid

    lanes = lax.iota(jnp.int32, 16)
    offs8 = lanes & 7

    gb = (g0, g1, g2)
    ob = (o0, o1, o2)
    pb = (p0, p1, p2)
    ib = (i0, i1, i2)
    sgb = (sg0, sg1, sg2)
    spb = (sp0, sp1, sp2)
    sob = (so0, so1, so2)

    def segment(ids_flat, out, T, start, n, nids):
        npb = T // 2
        n = jnp.int32(n)

        def pair_coords(k):
            p = start + k
            b = p // npb
            t = 2 * (p - b * npb)
            return b, t

        pltpu.sync_copy(ids_flat.at[pl.ds(16 * start, nids)],
                        idsbuf.at[pl.ds(0, nids)])

        def issue(k, par):
            b, t = pair_coords(k)
            ib[par][...] = (idsbuf[pl.ds(16 * k, 16)] << 3) + offs8
            pltpu.async_copy(wflat.at[ib[par]], gb[par], sgb[par])
            pltpu.async_copy(pos.at[pl.ds(t, 2), :], pb[par], spb[par])

        def drain_out(k, par):
            b, t = pair_coords(k)
            pltpu.make_async_copy(ob[par], out.at[b, pl.ds(t, 2)],
                                  sob[par]).wait()

        def process(k, par):
            b, t = pair_coords(k)
            pltpu.make_async_copy(wflat.at[ib[par]], gb[par], sgb[par]).wait()
            pltpu.make_async_copy(pos.at[pl.ds(t, 2), :], pb[par],
                                  spb[par]).wait()

            @pl.when(k >= _NB)
            def _():
                drain_out(k - _NB, par)

            rows = gb[par]
            pvs = pb[par]
            obuf = ob[par]

            def addp(c, _):
                cc = c * 16
                for tt in range(2):
                    for dh in range(8):
                        pv = pvs[tt, pl.ds(dh * 128 + cc, 16)]
                        for i in range(8):
                            obuf[tt, dh, i, pl.ds(cc, 16)] = (
                                rows[8 * tt + i, pl.ds(dh * 128 + cc, 16)]
                                + pv)
                return 0

            lax.fori_loop(0, 8, addp, 0)
            pltpu.async_copy(obuf, out.at[b, pl.ds(t, 2)], sob[par])

        @pl.when(n > 0)
        def _():
            issue(0, 0)

        def outer(kk, _):
            for par in range(_NB):
                k = kk * _NB + par

                @pl.when(k < n)
                def _(k=k, par=par):
                    kn = k + 1
                    parn = (par + 1) % _NB

                    @pl.when(kn < n)
                    def _():
                        issue(kn, parn)

                    process(k, par)
            return 0

        lax.fori_loop(0, (n + _NB - 1) // _NB, outer, 0)

        for par in range(_NB):
            klast = jnp.where((n - 1) % _NB == par, n - 1,
                              jnp.where((n - 2) % _NB == par, n - 2, n - 3))

            @pl.when(klast >= 0)
            def _(klast=klast, par=par):
                drain_out(klast, par)

    segment(prom_ids, prom_g, TP, wid * 32, 32, 512)
    ncode = B * (TC // 2)
    nc_extra = ncode - 32 * (ncode // 32)
    code_start = wid * (ncode // 32) + jnp.minimum(wid, nc_extra)
    code_n = ncode // 32 + jnp.where(wid < nc_extra, 1, 0)
    segment(code_ids, code_g, TC, code_start, code_n, 1056)


@jax.jit
def _sc_call(text_ids_pad, prom_ids_f, code_ids_f, text_table, wflat, pos,
             mvec):
    mesh = plsc.VectorSubcoreMesh(core_axis_name="c", subcore_axis_name="s",
                                  num_cores=2, num_subcores=16)
    params = pltpu.CompilerParams(use_tc_tiling_on_sc=False,
                                  needs_layout_passes=False)
    mt = pl.kernel(
        _sc_mt_body,
        mesh=mesh,
        compiler_params=params,
        out_type=[
            jax.ShapeDtypeStruct((B, LT, D), jnp.float32),
            jax.ShapeDtypeStruct((B * S * S,), jnp.float32),
        ],
        scratch_types=[
            pltpu.VMEM((16, D), jnp.float32),
            pltpu.VMEM((16, D), jnp.float32),
            pltpu.VMEM((16,), jnp.int32),
            pltpu.VMEM((B, 1024), jnp.float32),
            pltpu.VMEM((3600,), jnp.float32),
            pltpu.SemaphoreType.DMA,
        ],
    )
    text_e, mask_f = mt(text_ids_pad, text_table, pos, mvec)

    wave = pl.kernel(
        _sc_wave_body,
        mesh=mesh,
        compiler_params=params,
        out_type=[
            jax.ShapeDtypeStruct((B, TP, 8, NC, 128), jnp.float32),
            jax.ShapeDtypeStruct((B, TC, 8, NC, 128), jnp.float32),
        ],
        scratch_types=[
            pltpu.VMEM((16, D), jnp.float32),
            pltpu.VMEM((16, D), jnp.float32),
            pltpu.VMEM((16, D), jnp.float32),
            pltpu.VMEM((2, 8, NC, 128), jnp.float32),
            pltpu.VMEM((2, 8, NC, 128), jnp.float32),
            pltpu.VMEM((2, 8, NC, 128), jnp.float32),
            pltpu.VMEM((2, D), jnp.float32),
            pltpu.VMEM((2, D), jnp.float32),
            pltpu.VMEM((2, D), jnp.float32),
            pltpu.VMEM((16,), jnp.int32),
            pltpu.VMEM((16,), jnp.int32),
            pltpu.VMEM((16,), jnp.int32),
            pltpu.VMEM((1056,), jnp.int32),
            pltpu.SemaphoreType.DMA,
            pltpu.SemaphoreType.DMA,
            pltpu.SemaphoreType.DMA,
            pltpu.SemaphoreType.DMA,
            pltpu.SemaphoreType.DMA,
            pltpu.SemaphoreType.DMA,
            pltpu.SemaphoreType.DMA,
            pltpu.SemaphoreType.DMA,
            pltpu.SemaphoreType.DMA,
        ],
    )
    prom_g, code_g = wave(prom_ids_f, code_ids_f, wflat, pos)
    return text_e, prom_g, code_g, mask_f


def kernel(text, prom, code, text_table, wave_tables, pos_emb):
    text = jnp.where(text == -1, END_IND, text)
    prom_ids = jnp.where(prom == -1, END_IND, prom)
    code_in = jnp.where(code == -1, END_IND, code)
    text_ids = jnp.pad(text, ((0, 0), (1, 0)), constant_values=START_IND)
    text_ids = jnp.pad(text_ids, ((0, 0), (0, 1)), constant_values=END_IND)
    code_ids = jnp.pad(code_in, ((0, 0), (1, 0), (0, 0)),
                       constant_values=START_IND)
    code_ids = jnp.pad(code_ids, ((0, 0), (0, 1), (0, 0)),
                       constant_values=END_IND)
    gt = code_ids

    text_mask = jnp.pad((text_ids != END_IND)[:, :-1], ((0, 0), (1, 0)),
                        constant_values=True)
    prom_mask = jnp.pad((prom_ids != END_IND)[:, :-1, 0], ((0, 0), (1, 0)),
                        constant_values=True)
    code_mask = jnp.pad((code_ids != END_IND)[:, :-1, 0], ((0, 0), (1, 0)),
                        constant_values=True)
    m = jnp.concatenate((text_mask, prom_mask, code_mask), axis=1
                        ).astype(jnp.float32)
    gt_mask = code_mask

    text_ids_pad = jnp.pad(text_ids, ((0, 0), (0, 144 - LT)))
    prom_ids_f = prom_ids.reshape(B * TP * NC)
    code_ids_f = jnp.pad(code_ids.reshape(B * TC * NC), (0, 33024 - B * TC * NC))
    wflat = jnp.transpose(wave_tables, (1, 0, 2)).reshape(V * NC, D)
    mvec = jnp.pad(m, ((0, 0), (0, 1024 - S)))

    text_e, prom_g, code_g, mask_f = _sc_call(
        text_ids_pad, prom_ids_f, code_ids_f, text_table, wflat,
        pos_emb[:TC + 6], mvec)

    prom_e = prom_g.transpose(0, 1, 2, 4, 3).reshape(B, TP, D, NC)
    code_e = code_g.transpose(0, 1, 2, 4, 3).reshape(B, TC, D, NC)
    mask = mask_f.reshape(S, B, S).transpose(1, 0, 2)
    return (text_e, prom_e, code_e, mask, gt, gt_mask)
